# Initial kernel scaffold; baseline (speedup 1.0000x reference)
import jax
import jax.numpy as jnp
from jax.experimental import pallas as pl


def kernel(Gu, Gi, W_conv, W_dense, Q, edge_index, user, item):
    raise NotImplementedError("write your pallas kernel here")


# speedup vs baseline: 13.9035x; 13.9035x over previous
import functools

import jax
import jax.numpy as jnp
from jax import lax
from jax.experimental import pallas as pl
from jax.experimental.pallas import tpu as pltpu
from jax.experimental.pallas import tpu_sc as plsc

NU = 2000
NI = 8000
N = NU + NI
K = 128
E = 320000
B = 4096

SLABS = 32
DCW = 128
DCH = 79
EPAD = SLABS * DCH * DCW
TRASH = N
NPAD = 10240
RPT = NPAD // 16

_MESH = plsc.VectorSubcoreMesh(core_axis_name="c", subcore_axis_name="s")


@functools.partial(
    pl.kernel,
    out_type=jax.ShapeDtypeStruct((2, NPAD), jnp.float32),
    mesh=_MESH,
    scratch_types=[
        pltpu.VMEM((DCH, DCW), jnp.int32),
        pltpu.VMEM((DCW,), jnp.float32),
        pltpu.VMEM((RPT,), jnp.float32),
        pltpu.SemaphoreType.DMA,
        pltpu.VMEM_SHARED((NPAD,), jnp.float32),
    ],
)
def _deg_kernel(dst_hbm, deg_hbm, dslab, ones, zb, sem, deg_sh):
    cid = lax.axis_index("c")
    sid = lax.axis_index("s")

    def zr(i, carry):
        zb[pl.ds(i * 16, 16)] = jnp.zeros((16,), jnp.float32)
        return carry

    lax.fori_loop(0, RPT // 16, zr, 0)
    pltpu.sync_copy(zb, deg_sh.at[pl.ds(sid * RPT, RPT)])
    for i in range(DCW // 16):
        ones[pl.ds(i * 16, 16)] = jnp.ones((16,), jnp.float32)
    plsc.subcore_barrier()

    slab = cid * 16 + sid
    pltpu.sync_copy(dst_hbm.at[slab], dslab)

    def issue(j, carry):
        pltpu.async_copy(ones, deg_sh.at[dslab.at[j]], sem, add=True)
        return carry

    lax.fori_loop(0, DCH, issue, 0)

    def drain(j, carry):
        pltpu.make_async_copy(ones, deg_sh.at[dslab.at[0]], sem).wait()
        return carry

    lax.fori_loop(0, DCH, drain, 0)
    plsc.subcore_barrier()
    pltpu.sync_copy(deg_sh.at[pl.ds(sid * RPT, RPT)],
                    deg_hbm.at[cid, pl.ds(sid * RPT, RPT)])


@functools.partial(
    pl.kernel,
    out_type=(jax.ShapeDtypeStruct((NPAD, K), jnp.float32),
              jax.ShapeDtypeStruct((NPAD, K), jnp.float32)),
    mesh=_MESH,
    scratch_types=[
        pltpu.VMEM((2, DCW), jnp.int32),
        pltpu.VMEM((DCH, DCW), jnp.int32),
        pltpu.VMEM((DCW, K), jnp.float32),
        pltpu.VMEM((DCW, K), jnp.float32),
        pltpu.SemaphoreType.DMA,
        pltpu.SemaphoreType.DMA,
        pltpu.VMEM_SHARED((NPAD, K), jnp.float32),
    ],
)
def _scatter_kernel(src_hbm, dst_hbm, hp_hbm, out_a, out_b,
                    sring, dslab, rows0, rows1, gsem, isem, agg_sh):
    cid = lax.axis_index("c")
    sid = lax.axis_index("s")

    def zr(r, carry):
        for k in range(K // 16):
            rows0[r, pl.ds(k * 16, 16)] = jnp.zeros((16,), jnp.float32)
        return carry

    lax.fori_loop(0, DCW, zr, 0)
    for t in range(RPT // DCW):
        pltpu.sync_copy(rows0, agg_sh.at[pl.ds(sid * RPT + t * DCW, DCW)])
    plsc.subcore_barrier()

    slab = cid * 16 + sid
    pltpu.sync_copy(dst_hbm.at[slab], dslab)
    rows = (rows0, rows1)

    def sfetch(j, b):
        pltpu.async_copy(src_hbm.at[slab, j], sring.at[b], isem)

    def swait():
        pltpu.make_async_copy(src_hbm.at[slab, 0], sring.at[0], isem).wait()

    def gather(b, buf):
        pltpu.async_copy(hp_hbm.at[sring.at[b]], buf, gsem)

    def gwait(buf):
        pltpu.make_async_copy(hp_hbm.at[pl.ds(0, DCW)], buf, gsem).wait()

    sfetch(0, 0)
    swait()
    gather(0, rows0)
    sfetch(1, 1)

    def pairstep(p, carry):
        def one(j, ra, rb, slot_j, carry_unused=None):
            @pl.when(j < DCH)
            def _():
                gwait(ra)

                @pl.when(j < DCH - 1)
                def _():
                    swait()
                    gather(1 - slot_j, rb)

                @pl.when(j < DCH - 2)
                def _():
                    sfetch(j + 2, slot_j)

                pltpu.sync_copy(ra, agg_sh.at[dslab.at[j]], add=True)

        one(2 * p, rows0, rows1, 0)
        one(2 * p + 1, rows1, rows0, 1)
        return carry

    lax.fori_loop(0, (DCH + 1) // 2, pairstep, 0)
    plsc.subcore_barrier()

    @pl.when(cid == 0)
    def _():
        pltpu.sync_copy(agg_sh.at[pl.ds(sid * RPT, RPT)],
                        out_a.at[pl.ds(sid * RPT, RPT)])

    @pl.when(cid == 1)
    def _():
        pltpu.sync_copy(agg_sh.at[pl.ds(sid * RPT, RPT)],
                        out_b.at[pl.ds(sid * RPT, RPT)])


CHW = 128

@functools.partial(
    pl.kernel,
    out_type=jax.ShapeDtypeStruct((B,), jnp.float32),
    mesh=_MESH,
    scratch_types=[
        pltpu.VMEM((CHW,), jnp.int32),
        pltpu.VMEM((CHW,), jnp.int32),
        pltpu.VMEM((CHW, K), jnp.float32),
        pltpu.VMEM((CHW, K), jnp.float32),
        pltpu.VMEM((CHW,), jnp.float32),
        pltpu.SemaphoreType.DMA,
        pltpu.SemaphoreType.DMA,
    ],
)
def _dot_kernel(u_hbm, i_hbm, emb_hbm, t_hbm, out_hbm,
                uidx, iidx, buf_u, buf_i, ob, sem_u, sem_i):
    cid = lax.axis_index("c")
    sid = lax.axis_index("s")
    w = cid * 16 + sid
    pltpu.sync_copy(u_hbm.at[w], uidx)
    pltpu.sync_copy(i_hbm.at[w], iidx)
    cp_u = pltpu.async_copy(emb_hbm.at[uidx], buf_u, sem_u)
    cp_i = pltpu.async_copy(t_hbm.at[iidx], buf_i, sem_i)
    cp_u.wait()
    cp_i.wait()

    lanes = lax.iota(jnp.int32, 16)

    def grp(g, carry):
        accv = jnp.zeros((16,), jnp.float32)
        for rl in range(16):
            r = g * 16 + rl
            acc = jnp.zeros((16,), jnp.float32)
            for k in range(K // 16):
                acc = acc + buf_u[r, pl.ds(k * 16, 16)] * buf_i[r, pl.ds(k * 16, 16)]
            for sh in (8, 4, 2, 1):
                acc = acc + acc.at[lanes ^ sh].get(mode="promise_in_bounds")
            accv = jnp.where(lanes == rl, acc, accv)
        ob[pl.ds(g * 16, 16)] = accv
        return carry

    lax.fori_loop(0, CHW // 16, grp, 0)
    pltpu.sync_copy(ob, out_hbm.at[pl.ds(w * CHW, CHW)])


_BLK = 400


def _hprime_body(x_ref, w_ref, d0_ref, d1_ref, out_ref, dinv_ref):
    h = lax.dot_general(x_ref[...], w_ref[...], (((1,), (1,)), ((), ())),
                        preferred_element_type=jnp.float32)
    deg = d0_ref[...] + d1_ref[...]
    dinv = jnp.where(deg > 0, lax.rsqrt(jnp.maximum(deg, 1e-12)), 0.0)
    dinv_ref[...] = dinv
    out_ref[...] = h * dinv


def _hprime(x, w_conv, d0, d1):
    return pl.pallas_call(
        _hprime_body,
        grid=(N // _BLK,),
        in_specs=[
            pl.BlockSpec((_BLK, K), lambda b: (b, 0)),
            pl.BlockSpec((K, K), lambda b: (0, 0)),
            pl.BlockSpec((_BLK, 1), lambda b: (b, 0)),
            pl.BlockSpec((_BLK, 1), lambda b: (b, 0)),
        ],
        out_specs=[
            pl.BlockSpec((_BLK, K), lambda b: (b, 0)),
            pl.BlockSpec((_BLK, 1), lambda b: (b, 0)),
        ],
        out_shape=[
            jax.ShapeDtypeStruct((N, K), jnp.float32),
            jax.ShapeDtypeStruct((N, 1), jnp.float32),
        ],
    )(x, w_conv, d0, d1)


def _emb_body(a_ref, b_ref, dinv_ref, wd_ref, q_ref, emb_ref, t_ref):
    agg = jnp.maximum((a_ref[...] + b_ref[...]) * dinv_ref[...], 0.0)
    e = jnp.maximum(
        lax.dot_general(agg, wd_ref[...], (((1,), (1,)), ((), ())),
                        preferred_element_type=jnp.float32), 0.0)
    emb_ref[...] = e
    t_ref[...] = jnp.dot(e, q_ref[...], preferred_element_type=jnp.float32)


def _emb(agg_a, agg_b, dinv, w_dense, q):
    return pl.pallas_call(
        _emb_body,
        grid=(N // _BLK,),
        in_specs=[
            pl.BlockSpec((_BLK, K), lambda b: (b, 0)),
            pl.BlockSpec((_BLK, K), lambda b: (b, 0)),
            pl.BlockSpec((_BLK, 1), lambda b: (b, 0)),
            pl.BlockSpec((K, K), lambda b: (0, 0)),
            pl.BlockSpec((K, K), lambda b: (0, 0)),
        ],
        out_specs=[
            pl.BlockSpec((_BLK, K), lambda b: (b, 0)),
            pl.BlockSpec((_BLK, K), lambda b: (b, 0)),
        ],
        out_shape=[
            jax.ShapeDtypeStruct((N, K), jnp.float32),
            jax.ShapeDtypeStruct((N, K), jnp.float32),
        ],
    )(agg_a, agg_b, dinv, w_dense, q)


def kernel(Gu, Gi, W_conv, W_dense, Q, edge_index, user, item):
    x = jnp.concatenate([Gu, Gi], axis=0)
    src = edge_index[0].astype(jnp.int32)
    dst = edge_index[1].astype(jnp.int32)
    pad = EPAD - E
    srcp = jnp.concatenate([src, jnp.zeros((pad,), jnp.int32)])
    dstp = jnp.concatenate([dst, jnp.full((pad,), TRASH, jnp.int32)])

    src3 = srcp.reshape(SLABS, DCH, DCW)
    dst3 = dstp.reshape(SLABS, DCH, DCW)
    deg2 = _deg_kernel(dst3)
    d0 = deg2[0, :N].reshape(N, 1)
    d1 = deg2[1, :N].reshape(N, 1)
    hp, dinv = _hprime(x, W_conv, d0, d1)
    agg_a, agg_b = _scatter_kernel(src3, dst3, hp)
    emb, t = _emb(agg_a, agg_b, dinv, W_dense, Q)

    u2 = user.astype(jnp.int32).reshape(SLABS, CHW)
    i2 = (item.astype(jnp.int32) + NU).reshape(SLABS, CHW)
    return _dot_kernel(u2, i2, emb, t)


# speedup vs baseline: 24.9055x; 1.7913x over previous
import functools

import jax
import jax.numpy as jnp
from jax import lax
from jax.experimental import pallas as pl
from jax.experimental.pallas import tpu as pltpu
from jax.experimental.pallas import tpu_sc as plsc

NU = 2000
NI = 8000
N = NU + NI
K = 128
E = 320000
B = 4096

SLABS = 32
DCW = 128
DCH = 79
EPAD = SLABS * DCH * DCW
TRASH = N
NPAD = 10240
RPT = NPAD // 16

_MESH = plsc.VectorSubcoreMesh(core_axis_name="c", subcore_axis_name="s")


@functools.partial(
    pl.kernel,
    out_type=jax.ShapeDtypeStruct((2, NPAD), jnp.float32),
    mesh=_MESH,
    scratch_types=[
        pltpu.VMEM((DCH, DCW), jnp.int32),
        pltpu.VMEM((DCW,), jnp.float32),
        pltpu.VMEM((RPT,), jnp.float32),
        pltpu.SemaphoreType.DMA,
        pltpu.VMEM_SHARED((NPAD,), jnp.float32),
    ],
)
def _deg_kernel(dst_hbm, deg_hbm, dslab, ones, zb, sem, deg_sh):
    cid = lax.axis_index("c")
    sid = lax.axis_index("s")

    def zr(i, carry):
        zb[pl.ds(i * 16, 16)] = jnp.zeros((16,), jnp.float32)
        return carry

    lax.fori_loop(0, RPT // 16, zr, 0)
    pltpu.sync_copy(zb, deg_sh.at[pl.ds(sid * RPT, RPT)])
    for i in range(DCW // 16):
        ones[pl.ds(i * 16, 16)] = jnp.ones((16,), jnp.float32)
    plsc.subcore_barrier()

    slab = cid * 16 + sid
    pltpu.sync_copy(dst_hbm.at[slab], dslab)

    def issue(j, carry):
        pltpu.async_copy(ones, deg_sh.at[dslab.at[j]], sem, add=True)
        return carry

    lax.fori_loop(0, DCH, issue, 0)

    def drain(j, carry):
        pltpu.make_async_copy(ones, deg_sh.at[dslab.at[0]], sem).wait()
        return carry

    lax.fori_loop(0, DCH, drain, 0)
    plsc.subcore_barrier()
    pltpu.sync_copy(deg_sh.at[pl.ds(sid * RPT, RPT)],
                    deg_hbm.at[cid, pl.ds(sid * RPT, RPT)])


NCHUNK = E // DCW
C0 = 1650


def _range_for(worker_lo, worker_n, tile):
    q = worker_n // 16
    r = worker_n % 16
    start = worker_lo + q * tile + jnp.minimum(tile, r)
    count = q + jnp.where(tile < r, 1, 0)
    return start, count


@functools.partial(
    pl.kernel,
    out_type=(jax.ShapeDtypeStruct((NPAD, K), jnp.float32),
              jax.ShapeDtypeStruct((NPAD, K), jnp.float32)),
    mesh=_MESH,
    scratch_types=[
        pltpu.VMEM((2, DCW), jnp.int32),
        pltpu.VMEM((2, DCW), jnp.int32),
        pltpu.VMEM((DCW, K), jnp.float32),
        pltpu.VMEM((DCW, K), jnp.float32),
        pltpu.SemaphoreType.DMA,
        pltpu.SemaphoreType.DMA,
        pltpu.SemaphoreType.DMA,
        pltpu.VMEM_SHARED((NPAD, K), jnp.float32),
    ],
)
def _scatter_kernel(src_hbm, dst_hbm, hp_hbm, out_a, out_b,
                    sring, dring, rows0, rows1, gsem, ssem, dsem, agg_sh):
    cid = lax.axis_index("c")
    sid = lax.axis_index("s")

    def zr(r, carry):
        for k in range(K // 16):
            rows0[r, pl.ds(k * 16, 16)] = jnp.zeros((16,), jnp.float32)
        return carry

    lax.fori_loop(0, DCW, zr, 0)
    for t in range(RPT // DCW):
        pltpu.sync_copy(rows0, agg_sh.at[pl.ds(sid * RPT + t * DCW, DCW)])
    plsc.subcore_barrier()

    lo = jnp.where(cid == 0, 0, C0)
    n_core = jnp.where(cid == 0, C0, NCHUNK - C0)
    start, count = _range_for(lo, n_core, sid)

    def sfetch(g, b):
        pltpu.async_copy(src_hbm.at[g], sring.at[b], ssem)

    def swait():
        pltpu.make_async_copy(src_hbm.at[0], sring.at[0], ssem).wait()

    def dfetch(g, b):
        pltpu.async_copy(dst_hbm.at[g], dring.at[b], dsem)

    def dwait():
        pltpu.make_async_copy(dst_hbm.at[0], dring.at[0], dsem).wait()

    def gather(b, buf):
        pltpu.async_copy(hp_hbm.at[sring.at[b]], buf, gsem)

    def gwait(buf):
        pltpu.make_async_copy(hp_hbm.at[pl.ds(0, DCW)], buf, gsem).wait()

    @pl.when(count > 0)
    def _():
        sfetch(start, 0)
        dfetch(start, 0)
        swait()
        gather(0, rows0)

    @pl.when(count > 1)
    def _():
        sfetch(start + 1, 1)
        dfetch(start + 1, 1)

    def pairstep(p, carry):
        def one(j, ra, rb, slot_j):
            @pl.when(j < count)
            def _():
                gwait(ra)

                @pl.when(j < count - 1)
                def _():
                    swait()
                    gather(1 - slot_j, rb)

                @pl.when(j < count - 2)
                def _():
                    sfetch(start + j + 2, slot_j)

                dwait()
                pltpu.sync_copy(ra, agg_sh.at[dring.at[slot_j]], add=True)

                @pl.when(j < count - 2)
                def _():
                    dfetch(start + j + 2, slot_j)

        one(2 * p, rows0, rows1, 0)
        one(2 * p + 1, rows1, rows0, 1)
        return carry

    nmax = max(C0, NCHUNK - C0) // 16 + 1
    lax.fori_loop(0, (nmax + 1) // 2, pairstep, 0)
    plsc.subcore_barrier()

    @pl.when(cid == 0)
    def _():
        pltpu.sync_copy(agg_sh.at[pl.ds(sid * RPT, RPT)],
                        out_a.at[pl.ds(sid * RPT, RPT)])

    @pl.when(cid == 1)
    def _():
        pltpu.sync_copy(agg_sh.at[pl.ds(sid * RPT, RPT)],
                        out_b.at[pl.ds(sid * RPT, RPT)])


CHW = 128

@functools.partial(
    pl.kernel,
    out_type=jax.ShapeDtypeStruct((B,), jnp.float32),
    mesh=_MESH,
    scratch_types=[
        pltpu.VMEM((CHW,), jnp.int32),
        pltpu.VMEM((CHW,), jnp.int32),
        pltpu.VMEM((CHW, K), jnp.float32),
        pltpu.VMEM((CHW, K), jnp.float32),
        pltpu.VMEM((CHW,), jnp.float32),
        pltpu.SemaphoreType.DMA,
        pltpu.SemaphoreType.DMA,
    ],
)
def _dot_kernel(u_hbm, i_hbm, emb_hbm, t_hbm, out_hbm,
                uidx, iidx, buf_u, buf_i, ob, sem_u, sem_i):
    cid = lax.axis_index("c")
    sid = lax.axis_index("s")
    w = cid * 16 + sid
    pltpu.sync_copy(u_hbm.at[w], uidx)
    pltpu.sync_copy(i_hbm.at[w], iidx)
    cp_u = pltpu.async_copy(emb_hbm.at[uidx], buf_u, sem_u)
    cp_i = pltpu.async_copy(t_hbm.at[iidx], buf_i, sem_i)
    cp_u.wait()
    cp_i.wait()

    lanes = lax.iota(jnp.int32, 16)

    def grp(g, carry):
        accv = jnp.zeros((16,), jnp.float32)
        for rl in range(16):
            r = g * 16 + rl
            acc = jnp.zeros((16,), jnp.float32)
            for k in range(K // 16):
                acc = acc + buf_u[r, pl.ds(k * 16, 16)] * buf_i[r, pl.ds(k * 16, 16)]
            for sh in (8, 4, 2, 1):
                acc = acc + acc.at[lanes ^ sh].get(mode="promise_in_bounds")
            accv = jnp.where(lanes == rl, acc, accv)
        ob[pl.ds(g * 16, 16)] = accv
        return carry

    lax.fori_loop(0, CHW // 16, grp, 0)
    pltpu.sync_copy(ob, out_hbm.at[pl.ds(w * CHW, CHW)])


_BLK = 2000


def _hraw_body(x_ref, w_ref, out_ref):
    out_ref[...] = lax.dot_general(
        x_ref[...], w_ref[...], (((1,), (1,)), ((), ())),
        preferred_element_type=jnp.float32)


def _hraw(x, w_conv):
    return pl.pallas_call(
        _hraw_body,
        grid=(N // _BLK,),
        in_specs=[
            pl.BlockSpec((_BLK, K), lambda b: (b, 0)),
            pl.BlockSpec((K, K), lambda b: (0, 0)),
        ],
        out_specs=pl.BlockSpec((_BLK, K), lambda b: (b, 0)),
        out_shape=jax.ShapeDtypeStruct((N, K), jnp.float32),
    )(x, w_conv)


def _scale_body(h_ref, d0_ref, d1_ref, out_ref, dinv_ref):
    deg = d0_ref[...] + d1_ref[...]
    dinv = jnp.where(deg > 0, lax.rsqrt(jnp.maximum(deg, 1e-12)), 0.0)
    dinv_ref[...] = dinv
    out_ref[...] = h_ref[...] * dinv


def _hprime(h_raw, d0, d1):
    return pl.pallas_call(
        _scale_body,
        grid=(N // _BLK,),
        in_specs=[
            pl.BlockSpec((_BLK, K), lambda b: (b, 0)),
            pl.BlockSpec((_BLK, 1), lambda b: (b, 0)),
            pl.BlockSpec((_BLK, 1), lambda b: (b, 0)),
        ],
        out_specs=[
            pl.BlockSpec((_BLK, K), lambda b: (b, 0)),
            pl.BlockSpec((_BLK, 1), lambda b: (b, 0)),
        ],
        out_shape=[
            jax.ShapeDtypeStruct((N, K), jnp.float32),
            jax.ShapeDtypeStruct((N, 1), jnp.float32),
        ],
    )(h_raw, d0, d1)


def _emb_body(a_ref, b_ref, dinv_ref, wd_ref, q_ref, emb_ref, t_ref):
    agg = jnp.maximum((a_ref[...] + b_ref[...]) * dinv_ref[...], 0.0)
    e = jnp.maximum(
        lax.dot_general(agg, wd_ref[...], (((1,), (1,)), ((), ())),
                        preferred_element_type=jnp.float32), 0.0)
    emb_ref[...] = e
    t_ref[...] = jnp.dot(e, q_ref[...], preferred_element_type=jnp.float32)


def _emb(agg_a, agg_b, dinv, w_dense, q):
    return pl.pallas_call(
        _emb_body,
        grid=(N // _BLK,),
        in_specs=[
            pl.BlockSpec((_BLK, K), lambda b: (b, 0)),
            pl.BlockSpec((_BLK, K), lambda b: (b, 0)),
            pl.BlockSpec((_BLK, 1), lambda b: (b, 0)),
            pl.BlockSpec((K, K), lambda b: (0, 0)),
            pl.BlockSpec((K, K), lambda b: (0, 0)),
        ],
        out_specs=[
            pl.BlockSpec((_BLK, K), lambda b: (b, 0)),
            pl.BlockSpec((_BLK, K), lambda b: (b, 0)),
        ],
        out_shape=[
            jax.ShapeDtypeStruct((N, K), jnp.float32),
            jax.ShapeDtypeStruct((N, K), jnp.float32),
        ],
    )(agg_a, agg_b, dinv, w_dense, q)


def kernel(Gu, Gi, W_conv, W_dense, Q, edge_index, user, item):
    x = jnp.concatenate([Gu, Gi], axis=0)
    src = edge_index[0].astype(jnp.int32)
    dst = edge_index[1].astype(jnp.int32)
    dstp = jnp.concatenate(
        [dst, jnp.full((EPAD - E,), TRASH, jnp.int32)])

    h_raw = _hraw(x, W_conv)
    deg2 = _deg_kernel(dstp.reshape(SLABS, DCH, DCW))
    d0 = deg2[0, :N].reshape(N, 1)
    d1 = deg2[1, :N].reshape(N, 1)
    hp, dinv = _hprime(h_raw, d0, d1)
    agg_a, agg_b = _scatter_kernel(
        src.reshape(NCHUNK, DCW), dst.reshape(NCHUNK, DCW), hp)
    emb, t = _emb(agg_a, agg_b, dinv, W_dense, Q)

    u2 = user.astype(jnp.int32).reshape(SLABS, CHW)
    i2 = (item.astype(jnp.int32) + NU).reshape(SLABS, CHW)
    return _dot_kernel(u2, i2, emb, t)


# speedup vs baseline: 34.2594x; 1.3756x over previous
import functools

import jax
import jax.numpy as jnp
from jax import lax
from jax.experimental import pallas as pl
from jax.experimental.pallas import tpu as pltpu
from jax.experimental.pallas import tpu_sc as plsc

NU = 2000
NI = 8000
N = NU + NI
K = 128
E = 320000
B = 4096

SLABS = 32
DCW = 128
DCH = 79
EPAD = SLABS * DCH * DCW
TRASH = N
NPAD = 10112
RPT = NPAD // 16
DEGPAD = 10240
DRPT = DEGPAD // 16

_MESH = plsc.VectorSubcoreMesh(core_axis_name="c", subcore_axis_name="s")


@functools.partial(
    pl.kernel,
    out_type=jax.ShapeDtypeStruct((2, DEGPAD), jnp.float32),
    mesh=_MESH,
    scratch_types=[
        pltpu.VMEM((DCH, DCW), jnp.int32),
        pltpu.VMEM((DCW,), jnp.float32),
        pltpu.VMEM((DRPT,), jnp.float32),
        pltpu.SemaphoreType.DMA,
        pltpu.VMEM_SHARED((DEGPAD,), jnp.float32),
    ],
)
def _deg_kernel(dst_hbm, deg_hbm, dslab, ones, zb, sem, deg_sh):
    cid = lax.axis_index("c")
    sid = lax.axis_index("s")

    def zr(i, carry):
        zb[pl.ds(i * 16, 16)] = jnp.zeros((16,), jnp.float32)
        return carry

    lax.fori_loop(0, DRPT // 16, zr, 0)
    pltpu.sync_copy(zb, deg_sh.at[pl.ds(sid * DRPT, DRPT)])
    for i in range(DCW // 16):
        ones[pl.ds(i * 16, 16)] = jnp.ones((16,), jnp.float32)
    plsc.subcore_barrier()

    slab = cid * 16 + sid
    pltpu.sync_copy(dst_hbm.at[slab], dslab)

    def issue(j, carry):
        pltpu.async_copy(ones, deg_sh.at[dslab.at[j]], sem, add=True)
        return carry

    lax.fori_loop(0, DCH, issue, 0)

    def drain(j, carry):
        pltpu.make_async_copy(ones, deg_sh.at[dslab.at[0]], sem).wait()
        return carry

    lax.fori_loop(0, DCH, drain, 0)
    plsc.subcore_barrier()
    pltpu.sync_copy(deg_sh.at[pl.ds(sid * DRPT, DRPT)],
                    deg_hbm.at[cid, pl.ds(sid * DRPT, DRPT)])


NCHUNK = E // DCW
C0 = 1250


def _range_for(worker_lo, worker_n, tile):
    q = worker_n // 16
    r = worker_n % 16
    start = worker_lo + q * tile + jnp.minimum(tile, r)
    count = q + jnp.where(tile < r, 1, 0)
    return start, count


@functools.partial(
    pl.kernel,
    out_type=(jax.ShapeDtypeStruct((NPAD, K), jnp.float32),
              jax.ShapeDtypeStruct((NPAD, K), jnp.float32)),
    mesh=_MESH,
    scratch_types=[
        pltpu.VMEM((3, DCW), jnp.int32),
        pltpu.VMEM((3, DCW), jnp.int32),
        pltpu.VMEM((DCW, K), jnp.float32),
        pltpu.VMEM((DCW, K), jnp.float32),
        pltpu.VMEM((DCW, K), jnp.float32),
        pltpu.SemaphoreType.DMA,
        pltpu.SemaphoreType.DMA,
        pltpu.SemaphoreType.DMA,
        pltpu.SemaphoreType.DMA,
        pltpu.VMEM_SHARED((NPAD, K), jnp.float32),
    ],
)
def _scatter_kernel(src_hbm, dst_hbm, hp_hbm, out_a, out_b,
                    sring, dring, rows0, rows1, rows2,
                    gsem, ssem, dsem, csem, agg_sh):
    cid = lax.axis_index("c")
    sid = lax.axis_index("s")
    rows = (rows0, rows1, rows2)

    def zr(r, carry):
        for k in range(K // 16):
            rows0[r, pl.ds(k * 16, 16)] = jnp.zeros((16,), jnp.float32)
        return carry

    lax.fori_loop(0, DCW, zr, 0)
    for t in range(RPT // DCW):
        pltpu.sync_copy(rows0, agg_sh.at[pl.ds(sid * RPT + t * DCW, DCW)])
    pltpu.sync_copy(
        rows0.at[pl.ds(0, RPT - (RPT // DCW) * DCW)],
        agg_sh.at[pl.ds(sid * RPT + (RPT // DCW) * DCW,
                        RPT - (RPT // DCW) * DCW)])
    plsc.subcore_barrier()

    lo = jnp.where(cid == 0, 0, C0)
    n_core = jnp.where(cid == 0, C0, NCHUNK - C0)
    start, count = _range_for(lo, n_core, sid)

    def sfetch(j, b):
        pltpu.async_copy(src_hbm.at[start + j], sring.at[b], ssem)

    def swait():
        pltpu.make_async_copy(src_hbm.at[0], sring.at[0], ssem).wait()

    def dfetch(j, b):
        pltpu.async_copy(dst_hbm.at[start + j], dring.at[b], dsem)

    def dwait():
        pltpu.make_async_copy(dst_hbm.at[0], dring.at[0], dsem).wait()

    def gather(b, buf):
        pltpu.async_copy(hp_hbm.at[sring.at[b]], buf, gsem)

    def gwait(buf):
        pltpu.make_async_copy(hp_hbm.at[pl.ds(0, DCW)], buf, gsem).wait()

    def scat(b, buf):
        pltpu.async_copy(buf, agg_sh.at[dring.at[b]], csem, add=True)

    def scwait():
        pltpu.make_async_copy(rows0, agg_sh.at[dring.at[0]], csem).wait()

    for jj in range(3):
        @pl.when(count > jj)
        def _(jj=jj):
            sfetch(jj, jj)

    for jj in range(2):
        @pl.when(count > jj)
        def _(jj=jj):
            dfetch(jj, jj)

    @pl.when(count > 0)
    def _():
        swait()
        gather(0, rows0)

    @pl.when(count > 1)
    def _():
        swait()
        gather(1, rows1)

    def tristep(t, carry):
        for u in range(3):
            j = 3 * t + u

            @pl.when(j < count)
            def _(j=j, u=u):
                ra = rows[u]
                gwait(ra)
                dwait()
                scat(u, ra)

                @pl.when(j >= 1)
                def _():
                    scwait()

                @pl.when(j + 2 < count)
                def _():
                    swait()
                    gather((u + 2) % 3, rows[(u + 2) % 3])
                    dfetch(j + 2, (u + 2) % 3)

                @pl.when(j + 3 < count)
                def _():
                    sfetch(j + 3, u)

        return carry

    nmax = max(C0, NCHUNK - C0) // 16 + 1
    lax.fori_loop(0, (nmax + 2) // 3, tristep, 0)

    @pl.when(count > 0)
    def _():
        scwait()

    plsc.subcore_barrier()

    @pl.when(cid == 0)
    def _():
        pltpu.sync_copy(agg_sh.at[pl.ds(sid * RPT, RPT)],
                        out_a.at[pl.ds(sid * RPT, RPT)])

    @pl.when(cid == 1)
    def _():
        pltpu.sync_copy(agg_sh.at[pl.ds(sid * RPT, RPT)],
                        out_b.at[pl.ds(sid * RPT, RPT)])


CHW = 128

@functools.partial(
    pl.kernel,
    out_type=jax.ShapeDtypeStruct((B,), jnp.float32),
    mesh=_MESH,
    scratch_types=[
        pltpu.VMEM((CHW,), jnp.int32),
        pltpu.VMEM((CHW,), jnp.int32),
        pltpu.VMEM((CHW, K), jnp.float32),
        pltpu.VMEM((CHW, K), jnp.float32),
        pltpu.VMEM((CHW,), jnp.float32),
        pltpu.SemaphoreType.DMA,
        pltpu.SemaphoreType.DMA,
    ],
)
def _dot_kernel(u_hbm, i_hbm, emb_hbm, t_hbm, out_hbm,
                uidx, iidx, buf_u, buf_i, ob, sem_u, sem_i):
    cid = lax.axis_index("c")
    sid = lax.axis_index("s")
    w = cid * 16 + sid
    pltpu.sync_copy(u_hbm.at[w], uidx)
    pltpu.sync_copy(i_hbm.at[w], iidx)
    cp_u = pltpu.async_copy(emb_hbm.at[uidx], buf_u, sem_u)
    cp_i = pltpu.async_copy(t_hbm.at[iidx], buf_i, sem_i)
    cp_u.wait()
    cp_i.wait()

    lanes = lax.iota(jnp.int32, 16)

    def grp(g, carry):
        accv = jnp.zeros((16,), jnp.float32)
        for rl in range(16):
            r = g * 16 + rl
            acc = jnp.zeros((16,), jnp.float32)
            for k in range(K // 16):
                acc = acc + buf_u[r, pl.ds(k * 16, 16)] * buf_i[r, pl.ds(k * 16, 16)]
            for sh in (8, 4, 2, 1):
                acc = acc + acc.at[lanes ^ sh].get(mode="promise_in_bounds")
            accv = jnp.where(lanes == rl, acc, accv)
        ob[pl.ds(g * 16, 16)] = accv
        return carry

    lax.fori_loop(0, CHW // 16, grp, 0)
    pltpu.sync_copy(ob, out_hbm.at[pl.ds(w * CHW, CHW)])


_BLK = 2000


def _hraw_body(x_ref, w_ref, out_ref):
    out_ref[...] = lax.dot_general(
        x_ref[...], w_ref[...], (((1,), (1,)), ((), ())),
        preferred_element_type=jnp.float32)


def _hraw(x, w_conv):
    return pl.pallas_call(
        _hraw_body,
        grid=(N // _BLK,),
        in_specs=[
            pl.BlockSpec((_BLK, K), lambda b: (b, 0)),
            pl.BlockSpec((K, K), lambda b: (0, 0)),
        ],
        out_specs=pl.BlockSpec((_BLK, K), lambda b: (b, 0)),
        out_shape=jax.ShapeDtypeStruct((N, K), jnp.float32),
    )(x, w_conv)


def _scale_body(h_ref, d0_ref, d1_ref, out_ref, dinv_ref):
    deg = d0_ref[...] + d1_ref[...]
    dinv = jnp.where(deg > 0, lax.rsqrt(jnp.maximum(deg, 1e-12)), 0.0)
    dinv_ref[...] = dinv
    out_ref[...] = h_ref[...] * dinv


def _hprime(h_raw, d0, d1):
    return pl.pallas_call(
        _scale_body,
        grid=(N // _BLK,),
        in_specs=[
            pl.BlockSpec((_BLK, K), lambda b: (b, 0)),
            pl.BlockSpec((_BLK, 1), lambda b: (b, 0)),
            pl.BlockSpec((_BLK, 1), lambda b: (b, 0)),
        ],
        out_specs=[
            pl.BlockSpec((_BLK, K), lambda b: (b, 0)),
            pl.BlockSpec((_BLK, 1), lambda b: (b, 0)),
        ],
        out_shape=[
            jax.ShapeDtypeStruct((N, K), jnp.float32),
            jax.ShapeDtypeStruct((N, 1), jnp.float32),
        ],
    )(h_raw, d0, d1)


def _emb_body(a_ref, b_ref, dinv_ref, wd_ref, q_ref, emb_ref, t_ref):
    agg = jnp.maximum((a_ref[...] + b_ref[...]) * dinv_ref[...], 0.0)
    e = jnp.maximum(
        lax.dot_general(agg, wd_ref[...], (((1,), (1,)), ((), ())),
                        preferred_element_type=jnp.float32), 0.0)
    emb_ref[...] = e
    t_ref[...] = jnp.dot(e, q_ref[...], preferred_element_type=jnp.float32)


def _emb(agg_a, agg_b, dinv, w_dense, q):
    return pl.pallas_call(
        _emb_body,
        grid=(N // _BLK,),
        in_specs=[
            pl.BlockSpec((_BLK, K), lambda b: (b, 0)),
            pl.BlockSpec((_BLK, K), lambda b: (b, 0)),
            pl.BlockSpec((_BLK, 1), lambda b: (b, 0)),
            pl.BlockSpec((K, K), lambda b: (0, 0)),
            pl.BlockSpec((K, K), lambda b: (0, 0)),
        ],
        out_specs=[
            pl.BlockSpec((_BLK, K), lambda b: (b, 0)),
            pl.BlockSpec((_BLK, K), lambda b: (b, 0)),
        ],
        out_shape=[
            jax.ShapeDtypeStruct((N, K), jnp.float32),
            jax.ShapeDtypeStruct((N, K), jnp.float32),
        ],
    )(agg_a, agg_b, dinv, w_dense, q)


def kernel(Gu, Gi, W_conv, W_dense, Q, edge_index, user, item):
    x = jnp.concatenate([Gu, Gi], axis=0)
    src = edge_index[0].astype(jnp.int32)
    dst = edge_index[1].astype(jnp.int32)
    dstp = jnp.concatenate(
        [dst, jnp.full((EPAD - E,), TRASH, jnp.int32)])

    h_raw = _hraw(x, W_conv)
    deg2 = _deg_kernel(dstp.reshape(SLABS, DCH, DCW))
    d0 = deg2[0, :N].reshape(N, 1)
    d1 = deg2[1, :N].reshape(N, 1)
    hp, dinv = _hprime(h_raw, d0, d1)
    agg_a, agg_b = _scatter_kernel(
        src.reshape(NCHUNK, DCW), dst.reshape(NCHUNK, DCW), hp)
    emb, t = _emb(agg_a, agg_b, dinv, W_dense, Q)

    u2 = user.astype(jnp.int32).reshape(SLABS, CHW)
    i2 = (item.astype(jnp.int32) + NU).reshape(SLABS, CHW)
    return _dot_kernel(u2, i2, emb, t)
